# baseline (speedup 1.0000x reference)
import functools

import jax
import jax.numpy as jnp
from jax import lax
from jax.experimental import pallas as pl
from jax.experimental.pallas import tpu as pltpu
from jax.experimental.pallas import tpu_sc as plsc

_N = 10000
_D = 128
_M = 32
_K = 8
_DD = _D // _K


_GATHER_CHUNK = 80


def _sc_gather(x, neighbors):
    info = plsc.get_sparse_core_info()
    nc, ns = info.num_cores, info.num_subcores
    nw = nc * ns
    edges = neighbors.shape[0]
    per_w = edges // nw
    steps = per_w // _GATHER_CHUNK
    mesh = plsc.VectorSubcoreMesh(core_axis_name="c", subcore_axis_name="s")

    @functools.partial(
        pl.kernel,
        mesh=mesh,
        out_type=jax.ShapeDtypeStruct((edges, _D), jnp.float32),
        scratch_types=[
            pltpu.VMEM((_GATHER_CHUNK,), jnp.int32),
            pltpu.VMEM((_GATHER_CHUNK, _D), jnp.float32),
            pltpu.SemaphoreType.DMA,
        ],
    )
    def gather_kernel(x_hbm, idx_hbm, out_hbm, idx_v, rows_v, sem):
        wid = lax.axis_index("s") * nc + lax.axis_index("c")
        base = wid * per_w

        def body(i, _):
            off = pl.multiple_of(base + i * _GATHER_CHUNK, 8)
            pltpu.sync_copy(idx_hbm.at[pl.ds(off, _GATHER_CHUNK)], idx_v)
            pltpu.async_copy(x_hbm.at[idx_v], rows_v, sem).wait()
            pltpu.sync_copy(rows_v, out_hbm.at[pl.ds(off, _GATHER_CHUNK)])
            return ()

        lax.fori_loop(0, steps, body, (), unroll=False)

    return gather_kernel(x, neighbors)


_TILE = 400


def _segs():
    f32 = jnp.float32
    d_i = lax.broadcasted_iota(jnp.int32, (_D, _K), 0)
    c_i = lax.broadcasted_iota(jnp.int32, (_D, _K), 1)
    seg = (d_i // _DD == c_i).astype(f32)
    c_j = lax.broadcasted_iota(jnp.int32, (_K, _D), 0)
    d_j = lax.broadcasted_iota(jnp.int32, (_K, _D), 1)
    seg_t = (d_j // _DD == c_j).astype(f32)
    return seg, seg_t


def _mm(a, b):
    return lax.dot_general(a, b, (((1,), (0,)), ((), ())),
                           preferred_element_type=jnp.float32)


def _norm_cap(v, seg, seg_t):
    s = _mm(v * v, seg)
    inv = 1.0 / jnp.maximum(jnp.sqrt(s), 1e-12)
    return v * _mm(inv, seg_t)


def _norm_body(x_ref, o_ref):
    seg, seg_t = _segs()
    o_ref[...] = _norm_cap(x_ref[...], seg, seg_t)


def _tc_normalize(x):
    block = 2000
    return pl.pallas_call(
        _norm_body,
        grid=(_N // block,),
        in_specs=[pl.BlockSpec((block, _D), lambda i: (i, 0))],
        out_specs=pl.BlockSpec((block, _D), lambda i: (i, 0)),
        out_shape=jax.ShapeDtypeStruct((_N, _D), jnp.float32),
        compiler_params=pltpu.CompilerParams(
            dimension_semantics=("arbitrary",)),
    )(x)


def _routing_body(x_ref, z_ref, o_ref):
    t = x_ref.shape[0]
    seg, seg_t = _segs()
    mm = _mm

    def norm_cap(v):
        return _norm_cap(v, seg, seg_t)

    xn = x_ref[...]
    zn3 = z_ref[...]

    u = jnp.sum(zn3, axis=1) * (1.0 / _K) + xn
    u = norm_cap(u)

    for it in range(5):
        e = (zn3 * u[:, None, :]).reshape(t * _M, _D)
        logits_t = lax.dot_general(seg, e, (((0,), (1,)), ((), ())),
                                   preferred_element_type=jnp.float32)
        w = jnp.exp(logits_t)
        p = w / jnp.sum(w, axis=0, keepdims=True)
        p_full = lax.dot_general(p, seg_t, (((0,), (0,)), ((), ())),
                                 preferred_element_type=jnp.float32)
        u = jnp.sum(zn3 * p_full.reshape(t, _M, _D), axis=1) + xn
        if it < 4:
            u = norm_cap(u)

    o_ref[...] = u


def _tc_routing(x, z3):
    n = x.shape[0]
    grid = (n // _TILE,)
    return pl.pallas_call(
        _routing_body,
        grid=grid,
        in_specs=[
            pl.BlockSpec((_TILE, _D), lambda i: (i, 0)),
            pl.BlockSpec((_TILE, _M, _D), lambda i: (i, 0, 0)),
        ],
        out_specs=pl.BlockSpec((_TILE, _D), lambda i: (i, 0)),
        out_shape=jax.ShapeDtypeStruct((n, _D), jnp.float32),
        compiler_params=pltpu.CompilerParams(
            dimension_semantics=("arbitrary",)),
    )(x, z3)


_NCHUNK = 5


def kernel(x, neighbors, max_iter):
    xn = _tc_normalize(x)
    cn = _N // _NCHUNK
    nb2 = neighbors.reshape(_NCHUNK, cn * _M)
    outs = []
    for c in range(_NCHUNK):
        z = _sc_gather(xn, nb2[c])
        xs = lax.slice_in_dim(xn, c * cn, (c + 1) * cn, axis=0)
        outs.append(_tc_routing(xs, z.reshape(cn, _M, _D)))
    return jnp.concatenate(outs, axis=0)


# speedup vs baseline: 1.1351x; 1.1351x over previous
import functools

import jax
import jax.numpy as jnp
from jax import lax
from jax.experimental import pallas as pl
from jax.experimental.pallas import tpu as pltpu
from jax.experimental.pallas import tpu_sc as plsc

_N = 10000
_D = 128
_M = 32
_K = 8
_DD = _D // _K


_GATHER_CHUNK = 80


def _sc_gather(x, neighbors):
    info = plsc.get_sparse_core_info()
    nc, ns = info.num_cores, info.num_subcores
    nw = nc * ns
    edges = neighbors.shape[0]
    per_w = edges // nw
    steps = per_w // _GATHER_CHUNK
    mesh = plsc.VectorSubcoreMesh(core_axis_name="c", subcore_axis_name="s")

    @functools.partial(
        pl.kernel,
        mesh=mesh,
        out_type=jax.ShapeDtypeStruct((edges, _D), jnp.float32),
        scratch_types=[
            pltpu.VMEM((per_w,), jnp.int32),
            pltpu.VMEM((_GATHER_CHUNK, _D), jnp.float32),
            pltpu.VMEM((_GATHER_CHUNK, _D), jnp.float32),
            pltpu.SemaphoreType.DMA,
            pltpu.SemaphoreType.DMA,
            pltpu.SemaphoreType.DMA,
            pltpu.SemaphoreType.DMA,
        ],
    )
    def gather_kernel(x_hbm, idx_hbm, out_hbm, idx_v, rows0, rows1,
                      g0, g1, s0, s1):
        wid = lax.axis_index("s") * nc + lax.axis_index("c")
        base = pl.multiple_of(wid * per_w, 8)
        pltpu.sync_copy(idx_hbm.at[pl.ds(base, per_w)], idx_v)
        bufs, gsems, ssems = (rows0, rows1), (g0, g1), (s0, s1)

        def gstart(i):
            c = _GATHER_CHUNK
            return pltpu.async_copy(
                x_hbm.at[idx_v.at[pl.ds(i * c, c)]], bufs[i % 2],
                gsems[i % 2])

        def sstart(i):
            off = pl.multiple_of(base + i * _GATHER_CHUNK, 8)
            return pltpu.async_copy(
                bufs[i % 2], out_hbm.at[pl.ds(off, _GATHER_CHUNK)],
                ssems[i % 2])

        gh = {0: gstart(0)}
        sh = {}
        for i in range(steps):
            gh[i].wait()
            if i + 1 < steps:
                if i >= 1:
                    sh[i - 1].wait()
                gh[i + 1] = gstart(i + 1)
            sh[i] = sstart(i)
        sh[steps - 1].wait()
        if steps >= 2:
            sh[steps - 2].wait()

    return gather_kernel(x, neighbors)


_TILE = 400


def _segs():
    f32 = jnp.float32
    d_i = lax.broadcasted_iota(jnp.int32, (_D, _K), 0)
    c_i = lax.broadcasted_iota(jnp.int32, (_D, _K), 1)
    seg = (d_i // _DD == c_i).astype(f32)
    c_j = lax.broadcasted_iota(jnp.int32, (_K, _D), 0)
    d_j = lax.broadcasted_iota(jnp.int32, (_K, _D), 1)
    seg_t = (d_j // _DD == c_j).astype(f32)
    return seg, seg_t


def _mm(a, b):
    return lax.dot_general(a, b, (((1,), (0,)), ((), ())),
                           preferred_element_type=jnp.float32)


def _norm_cap(v, seg, seg_t):
    s = _mm(v * v, seg)
    inv = 1.0 / jnp.maximum(jnp.sqrt(s), 1e-12)
    return v * _mm(inv, seg_t)


def _norm_body(x_ref, o_ref):
    seg, seg_t = _segs()
    o_ref[...] = _norm_cap(x_ref[...], seg, seg_t)


def _tc_normalize(x):
    block = 2000
    return pl.pallas_call(
        _norm_body,
        grid=(_N // block,),
        in_specs=[pl.BlockSpec((block, _D), lambda i: (i, 0))],
        out_specs=pl.BlockSpec((block, _D), lambda i: (i, 0)),
        out_shape=jax.ShapeDtypeStruct((_N, _D), jnp.float32),
        compiler_params=pltpu.CompilerParams(
            dimension_semantics=("arbitrary",)),
    )(x)


def _routing_body(x_ref, z_ref, o_ref):
    t = x_ref.shape[0]
    seg, seg_t = _segs()
    mm = _mm

    def norm_cap(v):
        return _norm_cap(v, seg, seg_t)

    xn = x_ref[...]
    zn3 = z_ref[...]

    u = jnp.sum(zn3, axis=1) * (1.0 / _K) + xn
    u = norm_cap(u)

    for it in range(5):
        e = (zn3 * u[:, None, :]).reshape(t * _M, _D)
        logits = mm(e, seg)
        w = jnp.exp(logits)
        p = w / jnp.sum(w, axis=1, keepdims=True)
        p_full = mm(p, seg_t).reshape(t, _M, _D)
        u = jnp.sum(zn3 * p_full, axis=1) + xn
        if it < 4:
            u = norm_cap(u)

    o_ref[...] = u


def _tc_routing(x, z3):
    n = x.shape[0]
    grid = (n // _TILE,)
    return pl.pallas_call(
        _routing_body,
        grid=grid,
        in_specs=[
            pl.BlockSpec((_TILE, _D), lambda i: (i, 0)),
            pl.BlockSpec((_TILE, _M, _D), lambda i: (i, 0, 0)),
        ],
        out_specs=pl.BlockSpec((_TILE, _D), lambda i: (i, 0)),
        out_shape=jax.ShapeDtypeStruct((n, _D), jnp.float32),
        compiler_params=pltpu.CompilerParams(
            dimension_semantics=("arbitrary",)),
    )(x, z3)


_NCHUNK = 5


def kernel(x, neighbors, max_iter):
    xn = _tc_normalize(x)
    cn = _N // _NCHUNK
    nb2 = neighbors.reshape(_NCHUNK, cn * _M)
    outs = []
    for c in range(_NCHUNK):
        z = _sc_gather(xn, nb2[c])
        xs = lax.slice_in_dim(xn, c * cn, (c + 1) * cn, axis=0)
        outs.append(_tc_routing(xs, z.reshape(cn, _M, _D)))
    return jnp.concatenate(outs, axis=0)
